# speedup vs baseline: 24.4443x; 24.4443x over previous
import math

import jax
import jax.numpy as jnp
from jax.experimental import pallas as pl
from jax.experimental.pallas import tpu as pltpu

_VD = 16
_D = 16
_W = 15
_L = 256


def _dd_kernel(hva_ref, hvb_ref, c_ref, crow_ref, p_ref, mwt_ref,
               out_ref, phi23_ref):
    k1 = pl.program_id(0)

    c_all = c_ref[...]

    @pl.when(k1 == 0)
    def _build_phi23():
        for k2 in range(_D):
            phi23_ref[k2 * _D:(k2 + 1) * _D, :] = (
                c_all * c_all[k2:k2 + 1, :])

    s1 = hva_ref[0] + hvb_ref[0]
    s3 = s1[0:255, :] + s1[1:256, :]
    s2 = s3[0:239, :] + s3[16:255, :]
    s2p = jnp.concatenate(
        [s2, jnp.zeros((17, _L), jnp.float32)], axis=0)
    win = (s2p[:, 0:240] + s2p[:, 16:256]) * (1.0 / 16.0)

    base = phi23_ref[...] * (crow_ref[0] * p_ref[...])

    colg = jax.lax.broadcasted_iota(jnp.int32, (_VD, _L), 1)
    rowg = jax.lax.broadcasted_iota(jnp.int32, (_VD, _L), 0)
    tile_m = (colg % _VD == rowg).astype(jnp.float32)
    g = jnp.dot(mwt_ref[...], tile_m, preferred_element_type=jnp.float32)
    selp = jax.lax.broadcasted_iota(jnp.int32, (_L, _VD), 0) // _VD
    seli = jax.lax.broadcasted_iota(jnp.int32, (_L, _VD), 1)
    sel = (selp == seli).astype(jnp.float32)

    for k4 in range(_W):
        w = win[:, k4 * _VD:(k4 + 1) * _VD]
        xb = jnp.dot(w, g, preferred_element_type=jnp.float32)
        t = xb * (base * c_all[k4:k4 + 1, :])
        out_ref[0, :, k4 * _VD:(k4 + 1) * _VD] = jnp.dot(
            t, sel, preferred_element_type=jnp.float32)
    out_ref[0, :, _W * _VD:] = jnp.zeros((_L, _VD), jnp.float32)


@jax.jit
def kernel(hypervol, M_w, P):
    k_idx = jnp.arange(_D, dtype=jnp.float32)[:, None]
    per = jnp.arange(_L, dtype=jnp.float32)[None, :] + 2.0
    c_tab = jnp.cos((2.0 * math.pi) * k_idx / per)
    c_rows = c_tab.reshape(_D, 1, _L)

    hv2 = hypervol.reshape(_D, _D * _D, _D * _VD)
    p_flat = P.reshape(1, _L)
    mwt = M_w.T

    out = pl.pallas_call(
        _dd_kernel,
        grid=(_W,),
        in_specs=[
            pl.BlockSpec((1, _L, _L), lambda i: (i, 0, 0)),
            pl.BlockSpec((1, _L, _L), lambda i: (i + 1, 0, 0)),
            pl.BlockSpec((_D, _L), lambda i: (0, 0)),
            pl.BlockSpec((1, 1, _L), lambda i: (i, 0, 0)),
            pl.BlockSpec((1, _L), lambda i: (0, 0)),
            pl.BlockSpec((_VD, _VD), lambda i: (0, 0)),
        ],
        out_specs=pl.BlockSpec((1, _L, _L), lambda i: (i, 0, 0)),
        out_shape=jax.ShapeDtypeStruct((_W, _L, _L), jnp.float32),
        scratch_shapes=[pltpu.VMEM((_L, _L), jnp.float32)],
    )(hv2, hv2, c_tab, c_rows, p_flat, mwt)

    out5 = out.reshape(_W, _D, _D, _D, _VD)
    return out5[:, :_W, :_W, :_W, :].reshape(_W ** 4, _VD)
